# baseline (speedup 1.0000x reference)
import functools

import jax
import jax.numpy as jnp
from jax import lax
from jax.experimental import pallas as pl
from jax.experimental.pallas import tpu as pltpu
from jax.experimental.pallas import tpu_sc as plsc

_CACHE_FRAMES = 256
_NEW_FRAMES = 64
_TOK = 128
_D = 16 * 64

_NSUB = 16
_FRAMES_PER_SUB = _CACHE_FRAMES // _NSUB
_R = 16
_CPF = _TOK // _R
_JOBS = _FRAMES_PER_SUB * _CPF
_NSLOT = 4
_LOOK = 2


def _sc_store(idx32, kf2, vf2, kc2, vc2):
    mesh = plsc.VectorSubcoreMesh(core_axis_name="c", subcore_axis_name="s")

    @functools.partial(
        pl.kernel,
        out_type=(
            jax.ShapeDtypeStruct((_CACHE_FRAMES, _TOK, _D), jnp.float32),
            jax.ShapeDtypeStruct((_CACHE_FRAMES, _TOK, _D), jnp.float32),
        ),
        mesh=mesh,
        scratch_types=[
            pltpu.SMEM((_NEW_FRAMES,), jnp.int32),
            pltpu.SMEM((_CACHE_FRAMES,), jnp.int32),
            pltpu.VMEM((_NEW_FRAMES,), jnp.int32),
            pltpu.VMEM((_NSLOT, _R), jnp.int32),
            pltpu.VMEM((_NSLOT, _R, _D), jnp.float32),
            pltpu.SemaphoreType.DMA((_NSLOT,)),
            pltpu.SemaphoreType.DMA((_NSLOT,)),
        ],
    )
    def store(idx_h, kf_h, vf_h, kc_h, vc_h, ok_h, ov_h,
              idx_s, src_s, idx_v, rix, buf, in_sems, out_sems):
        core = lax.axis_index("c")
        sub = lax.axis_index("s")
        pltpu.sync_copy(idx_h, idx_v)

        @pl.loop(0, _NEW_FRAMES // 16)
        def _(g):
            vec = idx_v[pl.ds(g * 16, 16)]
            for t in range(16):
                idx_s[g * 16 + t] = vec[t]

        @pl.loop(0, _CACHE_FRAMES)
        def _(j):
            src_s[j] = -1

        @pl.loop(0, _NEW_FRAMES)
        def _(i):
            src_s[idx_s[i]] = i

        base = sub * _FRAMES_PER_SUB
        row_iota = lax.iota(jnp.int32, 16)

        def run(new_h, cache_h, out_h):
            def start_in(g, slot):
                frame = base + g // _CPF
                row0 = (g % _CPF) * _R
                s = src_s[frame]

                @pl.when(s >= 0)
                def _():
                    rix.at[slot][...] = s * _TOK + row0 + row_iota
                    pltpu.async_copy(new_h.at[rix.at[slot]], buf.at[slot],
                                     in_sems.at[slot])

                @pl.when(s < 0)
                def _():
                    rix.at[slot][...] = frame * _TOK + row0 + row_iota
                    pltpu.async_copy(cache_h.at[rix.at[slot]], buf.at[slot],
                                     in_sems.at[slot])

            def wait_in(slot):
                pltpu.make_async_copy(cache_h.at[rix.at[slot]], buf.at[slot],
                                      in_sems.at[slot]).wait()

            def wait_out(slot):
                pltpu.make_async_copy(buf.at[slot],
                                      out_h.at[0, pl.ds(0, _R)],
                                      out_sems.at[slot]).wait()

            for g0 in range(_LOOK):
                start_in(g0, g0 % _NSLOT)

            @pl.loop(0, _JOBS, step=_NSLOT)
            def _(it):
                for slot in range(_NSLOT):
                    g = it + slot
                    frame = base + g // _CPF
                    row0 = (g % _CPF) * _R
                    wait_in(slot)
                    pltpu.async_copy(buf.at[slot],
                                     out_h.at[frame, pl.ds(row0, _R)],
                                     out_sems.at[slot])
                    g2 = g + _LOOK
                    slot2 = (slot + _LOOK) % _NSLOT

                    @pl.when(g2 < _JOBS)
                    def _():
                        @pl.when(g2 >= _NSLOT)
                        def _():
                            wait_out(slot2)
                        start_in(g2, slot2)

            for slot in range(_NSLOT):
                wait_out(slot)

        @pl.when(core == 0)
        def _():
            run(kf_h, kc_h, ok_h)

        @pl.when(core == 1)
        def _():
            run(vf_h, vc_h, ov_h)

    return store(idx32, kf2, vf2, kc2, vc2)


def kernel(k, v, idx, k_cache, v_cache):
    idx32 = idx.astype(jnp.int32) % _CACHE_FRAMES
    out_k, out_v = _sc_store(
        idx32,
        k.reshape(_NEW_FRAMES * _TOK, _D),
        v.reshape(_NEW_FRAMES * _TOK, _D),
        k_cache.reshape(_CACHE_FRAMES * _TOK, _D),
        v_cache.reshape(_CACHE_FRAMES * _TOK, _D))
    return out_k.reshape(k_cache.shape), out_v.reshape(v_cache.shape)


# speedup vs baseline: 1.6397x; 1.6397x over previous
import functools

import jax
import jax.numpy as jnp
from jax import lax
from jax.experimental import pallas as pl
from jax.experimental.pallas import tpu as pltpu
from jax.experimental.pallas import tpu_sc as plsc

_CACHE_FRAMES = 256
_NEW_FRAMES = 64
_TOK = 128
_D = 16 * 64

_ANY = pl.ANY
_SMEM = pltpu.MemorySpace.SMEM


_NBUF = 16
_LOOKAHEAD = 12


def _tc_body(idx_s, new_h, cache_h, out_h, src_s, buf, in_sems, out_sems):
    def init(j, c):
        src_s[j] = -1
        return c
    lax.fori_loop(0, _CACHE_FRAMES, init, 0, unroll=8)

    def setmap(i, c):
        src_s[idx_s[i]] = i
        return c
    lax.fori_loop(0, _NEW_FRAMES, setmap, 0, unroll=8)

    def start_in(w, b):
        s = src_s[w]

        @pl.when(s >= 0)
        def _():
            pltpu.make_async_copy(new_h.at[s], buf.at[b], in_sems.at[b]).start()

        @pl.when(s < 0)
        def _():
            pltpu.make_async_copy(cache_h.at[w], buf.at[b], in_sems.at[b]).start()

    def prime(w, c):
        start_in(w, w % _NBUF)
        return c
    lax.fori_loop(0, _LOOKAHEAD, prime, 0)

    def step(w, c):
        b = w % _NBUF
        pltpu.make_async_copy(cache_h.at[0], buf.at[b], in_sems.at[b]).wait()
        pltpu.make_async_copy(buf.at[b], out_h.at[w], out_sems.at[b]).start()
        u = w + _LOOKAHEAD

        @pl.when(u < _CACHE_FRAMES)
        def _():
            bu = u % _NBUF

            @pl.when(u >= _NBUF)
            def _():
                pltpu.make_async_copy(buf.at[bu], out_h.at[0],
                                      out_sems.at[bu]).wait()
            start_in(u, bu)
        return c
    lax.fori_loop(0, _CACHE_FRAMES, step, 0)

    def drain(b, c):
        pltpu.make_async_copy(buf.at[b], out_h.at[0], out_sems.at[b]).wait()
        return c
    lax.fori_loop(0, _NBUF, drain, 0)


def _tc_store(idx32, new2, cache2):
    return pl.pallas_call(
        _tc_body,
        out_shape=jax.ShapeDtypeStruct((_CACHE_FRAMES, _TOK, _D), jnp.float32),
        in_specs=[
            pl.BlockSpec(memory_space=_SMEM),
            pl.BlockSpec(memory_space=_ANY),
            pl.BlockSpec(memory_space=_ANY),
        ],
        out_specs=pl.BlockSpec(memory_space=_ANY),
        scratch_shapes=[
            pltpu.SMEM((_CACHE_FRAMES,), jnp.int32),
            pltpu.VMEM((_NBUF, _TOK, _D), jnp.float32),
            pltpu.SemaphoreType.DMA((_NBUF,)),
            pltpu.SemaphoreType.DMA((_NBUF,)),
        ],
    )(idx32, new2, cache2)


_NTEC = 32
_FRAMES_PER_TEC = _CACHE_FRAMES // _NTEC
_R = 32
_CPF = _TOK // _R
_JOBS = _FRAMES_PER_TEC * _CPF
_NSLOT = 2


def _sc_store(idx32, new3, cache3):
    mesh = plsc.VectorSubcoreMesh(core_axis_name="c", subcore_axis_name="s")

    @functools.partial(
        pl.kernel,
        out_type=jax.ShapeDtypeStruct((_CACHE_FRAMES, _TOK, _D), jnp.float32),
        mesh=mesh,
        scratch_types=[
            pltpu.SMEM((_NEW_FRAMES,), jnp.int32),
            pltpu.SMEM((_CACHE_FRAMES,), jnp.int32),
            pltpu.VMEM((_NEW_FRAMES,), jnp.int32),
            pltpu.VMEM((_NSLOT, _R, _D), jnp.float32),
            pltpu.SemaphoreType.DMA((_NSLOT,)),
        ],
    )
    def store(idx_h, new_h, cache_h, out_h, idx_s, src_s, idx_v, buf, out_sems):
        core = lax.axis_index("c")
        sub = lax.axis_index("s")
        pltpu.sync_copy(idx_h, idx_v)

        @pl.loop(0, _NEW_FRAMES // 16)
        def _(g):
            vec = idx_v[pl.ds(g * 16, 16)]
            for t in range(16):
                idx_s[g * 16 + t] = vec[t]

        @pl.loop(0, _CACHE_FRAMES)
        def _(j):
            src_s[j] = -1

        @pl.loop(0, _NEW_FRAMES)
        def _(i):
            src_s[idx_s[i]] = i

        wid = core * 16 + sub
        base = wid * _FRAMES_PER_TEC

        def sync_in(g, slot):
            frame = base + g // _CPF
            row0 = (g % _CPF) * _R
            s = src_s[frame]

            @pl.when(s >= 0)
            def _():
                pltpu.sync_copy(new_h.at[s, pl.ds(row0, _R)], buf.at[slot])

            @pl.when(s < 0)
            def _():
                pltpu.sync_copy(cache_h.at[frame, pl.ds(row0, _R)],
                                buf.at[slot])

        def wait_out(slot):
            pltpu.make_async_copy(buf.at[slot], out_h.at[0, pl.ds(0, _R)],
                                  out_sems.at[slot]).wait()

        @pl.loop(0, _JOBS, step=_NSLOT)
        def _(it):
            for slot in range(_NSLOT):
                g = it + slot
                frame = base + g // _CPF
                row0 = (g % _CPF) * _R

                @pl.when(g >= _NSLOT)
                def _():
                    wait_out(slot)

                sync_in(g, slot)
                pltpu.async_copy(buf.at[slot],
                                 out_h.at[frame, pl.ds(row0, _R)],
                                 out_sems.at[slot])

        for slot in range(_NSLOT):
            wait_out(slot)

    return store(idx32, new3, cache3)


def kernel(k, v, idx, k_cache, v_cache):
    idx32 = idx.astype(jnp.int32) % _CACHE_FRAMES
    out_k = _tc_store(
        idx32,
        k.reshape(_NEW_FRAMES, _TOK, _D),
        k_cache.reshape(_CACHE_FRAMES, _TOK, _D))
    out_v = _sc_store(
        idx32,
        v.reshape(_NEW_FRAMES, _TOK, _D),
        v_cache.reshape(_CACHE_FRAMES, _TOK, _D))
    return out_k.reshape(k_cache.shape), out_v.reshape(v_cache.shape)
